# baseline (speedup 1.0000x reference)
import functools

import jax
import jax.numpy as jnp
from jax.experimental import pallas as pl
from jax.experimental.pallas import tpu as pltpu

_ROW_TILE = 512


def _cell_kernel(graph_ref, x_ref, hc_ref, c_ref, wct_ref, bias_ref,
                 h_out_ref, c_out_ref, cbuf_ref, xht_ref, xhtb_ref, lb_ref,
                 t1t_ref, sems):
    b = pl.program_id(0)
    nb = pl.num_programs(0)
    n = c_ref.shape[1]
    h = c_ref.shape[-1]
    din = x_ref.shape[-1]
    nt = n // _ROW_TILE
    prec = jax.lax.Precision.DEFAULT
    dims_tt = (((1,), (1,)), ((), ()))

    def chunk_copy(bb, i):
        rows = slice(i * _ROW_TILE, (i + 1) * _ROW_TILE)
        return pltpu.make_async_copy(graph_ref.at[bb, rows, :],
                                     cbuf_ref.at[i], sems.at[i])

    @pl.when(b == 0)
    def _():
        for i in range(nt):
            chunk_copy(0, i).start()

    def dot_l(small_t, l_tile):
        return jax.lax.dot_general(small_t, l_tile, dims_tt, precision=prec,
                                   preferred_element_type=jnp.float32)

    dot_w = functools.partial(jnp.dot, precision=prec,
                              preferred_element_type=jnp.float32)

    xht_ref[0:din, :] = x_ref[0].T
    xht_ref[din:, :] = hc_ref[0].T
    xhtb_ref[...] = xht_ref[...].astype(jnp.bfloat16)

    for i in range(nt):
        rows = slice(i * _ROW_TILE, (i + 1) * _ROW_TILE)
        chunk_copy(b, i).wait()
        l_bf = cbuf_ref[i].astype(jnp.bfloat16)
        lb_ref[rows, :] = l_bf
        t1t_ref[:, rows] = dot_l(xhtb_ref[...], l_bf).astype(jnp.bfloat16)

        @pl.when(b < nb - 1)
        def _():
            chunk_copy(b + 1, i).start()

    xht = xht_ref[...]
    t1t = t1t_ref[...]

    for i in range(nt):
        rows = slice(i * _ROW_TILE, (i + 1) * _ROW_TILE)
        t2t = 2.0 * dot_l(t1t, lb_ref[rows, :]) - xht[:, rows]

        combined_t = (
            dot_w(wct_ref[0], xhtb_ref[:, rows])
            + dot_w(wct_ref[1], t1t[:, rows])
            + dot_w(wct_ref[2], t2t.astype(jnp.bfloat16))
        )
        combined = combined_t.T + bias_ref[0]

        i_gate = jax.nn.sigmoid(combined[:, 0 * h:1 * h])
        f_gate = jax.nn.sigmoid(combined[:, 1 * h:2 * h])
        o_gate = jax.nn.sigmoid(combined[:, 2 * h:3 * h])
        g_gate = jnp.tanh(combined[:, 3 * h:4 * h])

        c_next = f_gate * c_ref[0, rows, :] + i_gate * g_gate
        c_out_ref[0, rows, :] = c_next
        h_out_ref[0, rows, :] = o_gate * jnp.tanh(c_next)


def kernel(input_tensor, graph, h_cur, c_cur, W1, b1, W2, b2, batch_size):
    B, N, Din = input_tensor.shape
    H = h_cur.shape[-1]
    K = W1.shape[0]
    F2 = Din + H

    wct = (jnp.concatenate([W1, W2], axis=1).transpose(0, 2, 1)
           .astype(jnp.bfloat16))
    bias = (b1 + b2).reshape(1, 4 * H)

    h_next, c_next = pl.pallas_call(
        _cell_kernel,
        grid=(B,),
        in_specs=[
            pl.BlockSpec(memory_space=pl.ANY),
            pl.BlockSpec((1, N, Din), lambda b: (b, 0, 0)),
            pl.BlockSpec((1, N, H), lambda b: (b, 0, 0)),
            pl.BlockSpec((1, N, H), lambda b: (b, 0, 0)),
            pl.BlockSpec((K, 4 * H, F2), lambda b: (0, 0, 0)),
            pl.BlockSpec((1, 4 * H), lambda b: (0, 0)),
        ],
        out_specs=[
            pl.BlockSpec((1, N, H), lambda b: (b, 0, 0)),
            pl.BlockSpec((1, N, H), lambda b: (b, 0, 0)),
        ],
        out_shape=[
            jax.ShapeDtypeStruct((B, N, H), jnp.float32),
            jax.ShapeDtypeStruct((B, N, H), jnp.float32),
        ],
        scratch_shapes=[
            pltpu.VMEM((N // _ROW_TILE, _ROW_TILE, N), jnp.float32),
            pltpu.VMEM((F2, N), jnp.float32),
            pltpu.VMEM((F2, N), jnp.bfloat16),
            pltpu.VMEM((N, N), jnp.bfloat16),
            pltpu.VMEM((F2, N), jnp.bfloat16),
            pltpu.SemaphoreType.DMA((N // _ROW_TILE,)),
        ],
    )(graph, input_tensor, h_cur, c_cur, wct, bias)
    return (h_next, c_next)


# speedup vs baseline: 1.0558x; 1.0558x over previous
import functools

import jax
import jax.numpy as jnp
from jax.experimental import pallas as pl
from jax.experimental.pallas import tpu as pltpu

_ROW_TILE = 512


def _cell_kernel(graph_ref, x_ref, hc_ref, c_ref, wct_ref, bias_ref, h_out_ref,
                 c_out_ref, xht_ref, xhtb_ref, lb_ref, t1t_ref):
    n = graph_ref.shape[1]
    h = c_ref.shape[-1]
    din = x_ref.shape[-1]
    prec = jax.lax.Precision.DEFAULT
    dims_tt = (((1,), (1,)), ((), ()))

    def dot_l(small_t, l_tile):
        return jax.lax.dot_general(small_t, l_tile, dims_tt, precision=prec,
                                   preferred_element_type=jnp.float32)

    dot_w = functools.partial(jnp.dot, precision=prec,
                              preferred_element_type=jnp.float32)

    xht_ref[0:din, :] = x_ref[0].T
    xht_ref[din:, :] = hc_ref[0].T
    xhtb_ref[...] = xht_ref[...].astype(jnp.bfloat16)

    for i in range(n // _ROW_TILE):
        rows = slice(i * _ROW_TILE, (i + 1) * _ROW_TILE)
        l_bf = graph_ref[0, rows, :].astype(jnp.bfloat16)
        lb_ref[rows, :] = l_bf
        t1t_ref[:, rows] = dot_l(xhtb_ref[...], l_bf).astype(jnp.bfloat16)

    xht = xht_ref[...]
    t1t = t1t_ref[...]

    for i in range(n // _ROW_TILE):
        rows = slice(i * _ROW_TILE, (i + 1) * _ROW_TILE)
        t2t = 2.0 * dot_l(t1t, lb_ref[rows, :]) - xht[:, rows]

        combined_t = (
            dot_w(wct_ref[0], xhtb_ref[:, rows])
            + dot_w(wct_ref[1], t1t[:, rows])
            + dot_w(wct_ref[2], t2t.astype(jnp.bfloat16))
        )
        combined = combined_t.T + bias_ref[0]

        i_gate = jax.nn.sigmoid(combined[:, 0 * h:1 * h])
        f_gate = jax.nn.sigmoid(combined[:, 1 * h:2 * h])
        o_gate = jax.nn.sigmoid(combined[:, 2 * h:3 * h])
        g_gate = jnp.tanh(combined[:, 3 * h:4 * h])

        c_next = f_gate * c_ref[0, rows, :] + i_gate * g_gate
        c_out_ref[0, rows, :] = c_next
        h_out_ref[0, rows, :] = o_gate * jnp.tanh(c_next)


def kernel(input_tensor, graph, h_cur, c_cur, W1, b1, W2, b2, batch_size):
    B, N, Din = input_tensor.shape
    H = h_cur.shape[-1]
    K = W1.shape[0]
    F2 = Din + H

    wct = (jnp.concatenate([W1, W2], axis=1).transpose(0, 2, 1)
           .astype(jnp.bfloat16))
    bias = (b1 + b2).reshape(1, 4 * H)

    h_next, c_next = pl.pallas_call(
        _cell_kernel,
        grid=(B,),
        in_specs=[
            pl.BlockSpec((1, N, N), lambda b: (b, 0, 0)),
            pl.BlockSpec((1, N, Din), lambda b: (b, 0, 0)),
            pl.BlockSpec((1, N, H), lambda b: (b, 0, 0)),
            pl.BlockSpec((1, N, H), lambda b: (b, 0, 0)),
            pl.BlockSpec((K, 4 * H, F2), lambda b: (0, 0, 0)),
            pl.BlockSpec((1, 4 * H), lambda b: (0, 0)),
        ],
        out_specs=[
            pl.BlockSpec((1, N, H), lambda b: (b, 0, 0)),
            pl.BlockSpec((1, N, H), lambda b: (b, 0, 0)),
        ],
        out_shape=[
            jax.ShapeDtypeStruct((B, N, H), jnp.float32),
            jax.ShapeDtypeStruct((B, N, H), jnp.float32),
        ],
        scratch_shapes=[
            pltpu.VMEM((F2, N), jnp.float32),
            pltpu.VMEM((F2, N), jnp.bfloat16),
            pltpu.VMEM((N, N), jnp.bfloat16),
            pltpu.VMEM((F2, N), jnp.bfloat16),
        ],
    )(graph, input_tensor, h_cur, c_cur, wct, bias)
    return (h_next, c_next)
